# baseline (speedup 1.0000x reference)
import functools

import jax
import jax.numpy as jnp
from jax import lax
from jax.experimental import pallas as pl
from jax.experimental.pallas import tpu as pltpu
from jax.experimental.pallas import tpu_sc as plsc

N = 10000
ROWS = 1000
GRID = N // ROWS
E = 320000
CH = 96
EROWSP = 3456
E_PAD = EROWSP * CH
NSUB = 16
NP = 10240
DRAIN = NP // NSUB
DCH = 64


def _leaky(e):
    return jnp.where(e > 0, e, 0.2 * e)


def _elu(x):
    return jnp.where(x > 0, x, jnp.exp(x) - 1.0)


def _dense_first_body(x_ref, w_ref, ac_ref, h_ref, aa_ref):
    h = jnp.dot(x_ref[...], w_ref[...], preferred_element_type=jnp.float32)
    h_ref[...] = h
    aa_ref[...] = jnp.dot(h, ac_ref[...], preferred_element_type=jnp.float32)


def _dense_bn_body(x_ref, st_ref, g_ref, be_ref, w_ref, ac_ref, h_ref, aa_ref):
    mu = st_ref[0]
    var = st_ref[1]
    rstd = jax.lax.rsqrt(var + 1e-5)
    xn = (x_ref[...] - mu[None, :]) * (rstd * g_ref[...])[None, :] + be_ref[...][None, :]
    xe = _elu(xn)
    h = jnp.dot(xe, w_ref[...], preferred_element_type=jnp.float32)
    h_ref[...] = h
    aa_ref[...] = jnp.dot(h, ac_ref[...], preferred_element_type=jnp.float32)


def _dense(x, W, Ac, stats=None, g=None, be=None):
    Din = x.shape[1]
    D = W.shape[1]
    if stats is None:
        return pl.pallas_call(
            _dense_first_body,
            grid=(GRID,),
            in_specs=[
                pl.BlockSpec((ROWS, Din), lambda i: (i, 0)),
                pl.BlockSpec((Din, D), lambda i: (0, 0)),
                pl.BlockSpec((D, 16), lambda i: (0, 0)),
            ],
            out_specs=[
                pl.BlockSpec((ROWS, D), lambda i: (i, 0)),
                pl.BlockSpec((ROWS, 16), lambda i: (i, 0)),
            ],
            out_shape=[
                jax.ShapeDtypeStruct((NP, D), jnp.float32),
                jax.ShapeDtypeStruct((NP, 16), jnp.float32),
            ],
        )(x, W, Ac)
    return pl.pallas_call(
        _dense_bn_body,
        grid=(GRID,),
        in_specs=[
            pl.BlockSpec((ROWS, Din), lambda i: (i, 0)),
            pl.BlockSpec((2, Din), lambda i: (0, 0)),
            pl.BlockSpec((Din,), lambda i: (0,)),
            pl.BlockSpec((Din,), lambda i: (0,)),
            pl.BlockSpec((Din, D), lambda i: (0, 0)),
            pl.BlockSpec((D, 16), lambda i: (0, 0)),
        ],
        out_specs=[
            pl.BlockSpec((ROWS, D), lambda i: (i, 0)),
            pl.BlockSpec((ROWS, 16), lambda i: (i, 0)),
        ],
        out_shape=[
            jax.ShapeDtypeStruct((NP, D), jnp.float32),
            jax.ShapeDtypeStruct((NP, 16), jnp.float32),
        ],
    )(x, stats, g, be, W, Ac)


def _combine_body(split, H, ch, num_ref, den_ref, h_ref, aa_ref, b_ref,
                  out_ref, st_ref):
    i = pl.program_id(0)
    D = H * ch
    asv = aa_ref[:, 0:8][:, 0:H]
    adv = aa_ref[:, 8:16][:, 0:H]
    wself = jnp.exp(_leaky(asv + adv))
    h3 = h_ref[...].reshape(ROWS, H, ch)
    if split == "edge":
        num3 = (num_ref[0] + num_ref[1]).reshape(ROWS, H, ch)
        den = den_ref[0, :, 0:H] + den_ref[1, :, 0:H]
    else:
        Hh = H // 2
        num3 = jnp.concatenate(
            [num_ref[0].reshape(ROWS, Hh, ch), num_ref[1].reshape(ROWS, Hh, ch)],
            axis=1)
        den = jnp.concatenate([den_ref[0, :, 0:Hh], den_ref[1, :, 0:Hh]], axis=1)
    den = den + wself
    num3 = num3 + wself[:, :, None] * h3
    out = num3 / den[:, :, None]
    out = out.reshape(ROWS, D) + b_ref[...][None, :]
    out_ref[...] = out

    @pl.when(i == 0)
    def _():
        st_ref[...] = jnp.zeros_like(st_ref)

    s = jnp.sum(out, axis=0)
    s2 = jnp.sum(out * out, axis=0)
    st_ref[0] += s
    st_ref[1] += s2

    @pl.when(i == GRID - 1)
    def _():
        mu = st_ref[0] / N
        st_ref[0] = mu
        st_ref[1] = st_ref[1] / N - mu * mu


def _combine(split, H, ch, num, den, h, aa, b):
    D = H * ch
    Dp = num.shape[2]
    return pl.pallas_call(
        functools.partial(_combine_body, split, H, ch),
        grid=(GRID,),
        in_specs=[
            pl.BlockSpec((2, ROWS, Dp), lambda i: (0, i, 0)),
            pl.BlockSpec((2, ROWS, 16), lambda i: (0, i, 0)),
            pl.BlockSpec((ROWS, D), lambda i: (i, 0)),
            pl.BlockSpec((ROWS, 16), lambda i: (i, 0)),
            pl.BlockSpec((D,), lambda i: (0,)),
        ],
        out_specs=[
            pl.BlockSpec((ROWS, D), lambda i: (i, 0)),
            pl.BlockSpec((2, D), lambda i: (0, 0)),
        ],
        out_shape=[
            jax.ShapeDtypeStruct((N, D), jnp.float32),
            jax.ShapeDtypeStruct((2, D), jnp.float32),
        ],
    )(num, den, h, aa, b)


def _final_body(num_ref, den_ref, h_ref, aa_ref, b_ref, out_ref):
    asv = aa_ref[:, 0:1]
    adv = aa_ref[:, 8:9]
    wself = jnp.exp(_leaky(asv + adv))
    den = den_ref[0, :, 0:1] + den_ref[1, :, 0:1] + wself
    num = num_ref[0] + num_ref[1] + wself * h_ref[...]
    out = num / den + b_ref[...][None, :]
    m = jnp.max(out, axis=1, keepdims=True)
    lse = m + jnp.log(jnp.sum(jnp.exp(out - m), axis=1, keepdims=True))
    out_ref[...] = out - lse


def _final(num, den, h, aa, b):
    return pl.pallas_call(
        _final_body,
        grid=(GRID,),
        in_specs=[
            pl.BlockSpec((2, ROWS, 16), lambda i: (0, i, 0)),
            pl.BlockSpec((2, ROWS, 16), lambda i: (0, i, 0)),
            pl.BlockSpec((ROWS, 16), lambda i: (i, 0)),
            pl.BlockSpec((ROWS, 16), lambda i: (i, 0)),
            pl.BlockSpec((16,), lambda i: (0,)),
        ],
        out_specs=pl.BlockSpec((ROWS, 16), lambda i: (i, 0)),
        out_shape=jax.ShapeDtypeStruct((N, 16), jnp.float32),
    )(num, den, h, aa, b)


def _edge_sc_body(split, Dp, head_of_block, n_chunks,
                  em_ref, h_ref, asn_ref, adn_ref,
                  num_ref, den_ref,
                  ei0, ei1, asb0, asb1, adb0, adb1,
                  hb0, hb1, wb0, wb1,
                  acc, dacc,
                  s_as0, s_as1, s_ad0, s_ad1, s_h0, s_h1,
                  s_sc0, s_sc1, s_sd0, s_sd1):
    EI = (ei0, ei1)
    ASB = (asb0, asb1)
    ADB = (adb0, adb1)
    HB = (hb0, hb1)
    WB = (wb0, wb1)
    S_AS = (s_as0, s_as1)
    S_AD = (s_ad0, s_ad1)
    S_H = (s_h0, s_h1)
    S_SC = (s_sc0, s_sc1)
    S_SD = (s_sd0, s_sd1)
    c = lax.axis_index("c")
    s = lax.axis_index("s")

    zb = hb0.at[pl.ds(0, DCH)]
    zd = wb0.at[pl.ds(0, DCH)]

    @pl.loop(0, DCH)
    def _z(i):
        for j in range(Dp // 16):
            zb[i, pl.ds(j * 16, 16)] = jnp.zeros((16,), jnp.float32)
        zd[i, pl.ds(0, 16)] = jnp.zeros((16,), jnp.float32)

    for t in range(DRAIN // DCH):
        base = pl.multiple_of(s * DRAIN + t * DCH, DCH)
        pltpu.sync_copy(zb, acc.at[pl.ds(base, DCH)])
        pltpu.sync_copy(zd, dacc.at[pl.ds(base, DCH)])
    plsc.subcore_barrier()

    if split == "edge":
        row_base = c * (EROWSP // 2) + s * n_chunks
        h_v, asn_v, adn_v = h_ref, asn_ref, adn_ref
    else:
        row_base = s * n_chunks
        h_v = h_ref.at[c]
        asn_v = asn_ref.at[c]
        adn_v = adn_ref.at[c]

    def load_idx(row, b):
        pltpu.sync_copy(em_ref.at[row], EI[b])

    def gather_descs(b):
        return (
            pltpu.make_async_copy(asn_v.at[EI[b].at[0]], ASB[b], S_AS[b]),
            pltpu.make_async_copy(adn_v.at[EI[b].at[1]], ADB[b], S_AD[b]),
            pltpu.make_async_copy(h_v.at[EI[b].at[0]], HB[b], S_H[b]),
        )

    def start_gathers(b):
        for d in gather_descs(b):
            d.start()

    def wait_gathers(b):
        for d in gather_descs(b):
            d.wait()

    def compute(b):
        asb, adb, hbuf, wbuf = ASB[b], ADB[b], HB[b], WB[b]

        @pl.loop(0, CH)
        def _row(r):
            e = asb[r] + adb[r]
            e = jnp.where(e > 0.0, e, 0.2 * e)
            w = jnp.exp(e)
            wbuf[r] = w
            for j in range(Dp // 16):
                ws = w[head_of_block[j]]
                hbuf[r, pl.ds(j * 16, 16)] = hbuf[r, pl.ds(j * 16, 16)] * ws

    def scatter(b):
        d1 = pltpu.async_copy(HB[b], acc.at[EI[b].at[1]], S_SC[b], add=True)
        d2 = pltpu.async_copy(WB[b], dacc.at[EI[b].at[1]], S_SD[b], add=True)
        d1.wait()
        d2.wait()

    load_idx(row_base, 0)
    start_gathers(0)
    load_idx(row_base + 1, 1)
    start_gathers(1)

    nh = n_chunks // 2

    @pl.loop(0, nh - 1)
    def _pair(kk):
        k0 = row_base + 2 * kk
        for b in (0, 1):
            wait_gathers(b)
            compute(b)
            scatter(b)
            load_idx(k0 + b + 2, b)
            start_gathers(b)

    for b in (0, 1):
        wait_gathers(b)
        compute(b)
        scatter(b)

    plsc.subcore_barrier()

    db = (hb0.at[pl.ds(0, DCH)], hb1.at[pl.ds(0, DCH)])
    dd = (wb0.at[pl.ds(0, DCH)], wb1.at[pl.ds(0, DCH)])
    for t in range(DRAIN // DCH):
        b = t % 2
        base = pl.multiple_of(s * DRAIN + t * DCH, DCH)
        pltpu.sync_copy(acc.at[pl.ds(base, DCH)], db[b])
        pltpu.sync_copy(db[b], num_ref.at[c].at[pl.ds(base, DCH)])
        pltpu.sync_copy(dacc.at[pl.ds(base, DCH)], dd[b])
        pltpu.sync_copy(dd[b], den_ref.at[c].at[pl.ds(base, DCH)])


def _edge_sc(split, H, ch, h, asn, adn, em):
    D = H * ch
    Dp = D if split == "edge" else D // 2
    if split == "edge":
        head_of_block = [min(j * 16 // ch, H - 1) for j in range(Dp // 16)]
        n_chunks = EROWSP // 2 // NSUB
    else:
        head_of_block = [j * 16 // ch for j in range(Dp // 16)]
        n_chunks = EROWSP // NSUB
    body = functools.partial(_edge_sc_body, split, Dp, head_of_block, n_chunks)
    mesh = plsc.VectorSubcoreMesh(core_axis_name="c", subcore_axis_name="s")
    kfn = pl.kernel(
        body,
        out_type=[
            jax.ShapeDtypeStruct((2, NP, Dp), jnp.float32),
            jax.ShapeDtypeStruct((2, NP, 16), jnp.float32),
        ],
        mesh=mesh,
        compiler_params=pltpu.CompilerParams(use_tc_tiling_on_sc=False),
        scratch_types=(
            [pltpu.VMEM((2, CH), jnp.int32)] * 2
            + [pltpu.VMEM((CH, 16), jnp.float32)] * 4
            + [pltpu.VMEM((CH, Dp), jnp.float32)] * 2
            + [pltpu.VMEM((CH, 16), jnp.float32)] * 2
            + [
                pltpu.VMEM_SHARED((NP, Dp), jnp.float32),
                pltpu.VMEM_SHARED((NP, 16), jnp.float32),
            ]
            + [pltpu.SemaphoreType.DMA] * 10
        ),
    )
    return kfn(em, h, asn, adn)


def _edges_jnp(h, asn, adn, src, dst, H, ch, split):
    D = H * ch
    e = asn[src] + adn[dst]
    w = jnp.exp(_leaky(e))
    den = jax.ops.segment_sum(w, dst, num_segments=N)
    msg = h[src].reshape(E, H, ch) * w[..., None]
    num = jax.ops.segment_sum(msg, dst, num_segments=N).reshape(N, D)
    den16 = jnp.zeros((N, 16), jnp.float32)
    if split == "edge":
        den16 = den16.at[:, 0:H].set(den)
        num_p = jnp.stack([num, jnp.zeros_like(num)])
        den_p = jnp.stack([den16, jnp.zeros_like(den16)])
    else:
        Hh = H // 2
        num_p = jnp.stack([num[:, : D // 2], num[:, D // 2:]])
        den_p = jnp.stack([
            den16.at[:, 0:Hh].set(den[:, 0:Hh]),
            den16.at[:, 0:Hh].set(den[:, Hh:]),
        ])
    return num_p, den_p


def _build_ac(a_s, a_d, D):
    H, ch = a_s.shape
    A = jnp.zeros((D, 16), jnp.float32)
    for hh in range(H):
        A = A.at[hh * ch:(hh + 1) * ch, hh].set(a_s[hh])
        A = A.at[hh * ch:(hh + 1) * ch, 8 + hh].set(a_d[hh])
    return A


def kernel(x, edge_index, W1, a1s, a1d, b1, g1, be1, W2, a2s, a2d, b2, g2, be2,
           W3, a3s, a3d, b3):
    src = edge_index[0]
    dst = edge_index[1]
    pad = jnp.full((E_PAD - E,), NP - 1, jnp.int32)
    srcp = jnp.concatenate([src, pad]).reshape(EROWSP, CH)
    dstp = jnp.concatenate([dst, pad]).reshape(EROWSP, CH)
    em = jnp.stack([srcp, dstp], axis=1)

    h1, aa1 = _dense(x, W1, _build_ac(a1s, a1d, 128))
    asn = jnp.pad(aa1[:, 0:8], ((0, 0), (0, 8)))
    adn = jnp.pad(aa1[:, 8:16], ((0, 0), (0, 8)))
    num, den = _edge_sc("edge", 8, 16, h1, asn, adn, em)
    out1, st1 = _combine("edge", 8, 16, num, den, h1, aa1, b1)

    h2, aa2 = _dense(out1, W2, _build_ac(a2s, a2d, 256), st1, g1, be1)
    h2s = jnp.stack([h2[:, 0:128], h2[:, 128:256]])
    asn2 = jnp.stack([jnp.pad(aa2[:, 0:4], ((0, 0), (0, 12))),
                      jnp.pad(aa2[:, 4:8], ((0, 0), (0, 12)))])
    adn2 = jnp.stack([jnp.pad(aa2[:, 8:12], ((0, 0), (0, 12))),
                      jnp.pad(aa2[:, 12:16], ((0, 0), (0, 12)))])
    num, den = _edge_sc("chan", 8, 32, h2s, asn2, adn2, em)
    out2, st2 = _combine("chan", 8, 32, num, den, h2, aa2, b2)

    h3, aa3 = _dense(out2, W3, _build_ac(a3s, a3d, 16), st2, g2, be2)
    asn = jnp.pad(aa3[:, 0:1], ((0, 0), (0, 15)))
    adn = jnp.pad(aa3[:, 8:9], ((0, 0), (0, 15)))
    num, den = _edge_sc("edge", 1, 16, h3, asn, adn, em)
    return _final(num, den, h3, aa3, b3)


# speedup vs baseline: 1.3611x; 1.3611x over previous
import functools

import jax
import jax.numpy as jnp
from jax import lax
from jax.experimental import pallas as pl
from jax.experimental.pallas import tpu as pltpu
from jax.experimental.pallas import tpu_sc as plsc

N = 10000
ROWS = 1000
GRID = N // ROWS
E = 320000
CH = 96
EROWSP = 3456
E_PAD = EROWSP * CH
NSUB = 16
NP = 10240
DRAIN = NP // NSUB
DCH = 64


def _leaky(e):
    return jnp.where(e > 0, e, 0.2 * e)


def _elu(x):
    return jnp.where(x > 0, x, jnp.exp(x) - 1.0)


def _dense_first_body(x_ref, w_ref, ac_ref, h_ref, aa_ref):
    h = jnp.dot(x_ref[...], w_ref[...], preferred_element_type=jnp.float32)
    h_ref[...] = h
    aa_ref[...] = jnp.dot(h, ac_ref[...], preferred_element_type=jnp.float32)


def _dense_bn_body(x_ref, st_ref, g_ref, be_ref, w_ref, ac_ref, h_ref, aa_ref):
    mu = st_ref[0]
    var = st_ref[1]
    rstd = jax.lax.rsqrt(var + 1e-5)
    xn = (x_ref[...] - mu[None, :]) * (rstd * g_ref[...])[None, :] + be_ref[...][None, :]
    xe = _elu(xn)
    h = jnp.dot(xe, w_ref[...], preferred_element_type=jnp.float32)
    h_ref[...] = h
    aa_ref[...] = jnp.dot(h, ac_ref[...], preferred_element_type=jnp.float32)


def _dense(x, W, Ac, stats=None, g=None, be=None):
    Din = x.shape[1]
    D = W.shape[1]
    if stats is None:
        return pl.pallas_call(
            _dense_first_body,
            grid=(GRID,),
            in_specs=[
                pl.BlockSpec((ROWS, Din), lambda i: (i, 0)),
                pl.BlockSpec((Din, D), lambda i: (0, 0)),
                pl.BlockSpec((D, 16), lambda i: (0, 0)),
            ],
            out_specs=[
                pl.BlockSpec((ROWS, D), lambda i: (i, 0)),
                pl.BlockSpec((ROWS, 16), lambda i: (i, 0)),
            ],
            out_shape=[
                jax.ShapeDtypeStruct((NP, D), jnp.float32),
                jax.ShapeDtypeStruct((NP, 16), jnp.float32),
            ],
        )(x, W, Ac)
    return pl.pallas_call(
        _dense_bn_body,
        grid=(GRID,),
        in_specs=[
            pl.BlockSpec((ROWS, Din), lambda i: (i, 0)),
            pl.BlockSpec((2, Din), lambda i: (0, 0)),
            pl.BlockSpec((Din,), lambda i: (0,)),
            pl.BlockSpec((Din,), lambda i: (0,)),
            pl.BlockSpec((Din, D), lambda i: (0, 0)),
            pl.BlockSpec((D, 16), lambda i: (0, 0)),
        ],
        out_specs=[
            pl.BlockSpec((ROWS, D), lambda i: (i, 0)),
            pl.BlockSpec((ROWS, 16), lambda i: (i, 0)),
        ],
        out_shape=[
            jax.ShapeDtypeStruct((NP, D), jnp.float32),
            jax.ShapeDtypeStruct((NP, 16), jnp.float32),
        ],
    )(x, stats, g, be, W, Ac)


def _combine_body(split, H, ch, num_ref, den_ref, h_ref, aa_ref, b_ref,
                  out_ref, st_ref):
    i = pl.program_id(0)
    D = H * ch
    asv = aa_ref[:, 0:8][:, 0:H]
    adv = aa_ref[:, 8:16][:, 0:H]
    wself = jnp.exp(_leaky(asv + adv))
    h3 = h_ref[...].reshape(ROWS, H, ch)
    if split == "edge":
        num3 = (num_ref[0] + num_ref[1]).reshape(ROWS, H, ch)
        den = den_ref[0, :, 0:H] + den_ref[1, :, 0:H]
    else:
        Hh = H // 2
        num3 = jnp.concatenate(
            [num_ref[0].reshape(ROWS, Hh, ch), num_ref[1].reshape(ROWS, Hh, ch)],
            axis=1)
        den = jnp.concatenate([den_ref[0, :, 0:Hh], den_ref[1, :, 0:Hh]], axis=1)
    den = den + wself
    num3 = num3 + wself[:, :, None] * h3
    out = num3 / den[:, :, None]
    out = out.reshape(ROWS, D) + b_ref[...][None, :]
    out_ref[...] = out

    @pl.when(i == 0)
    def _():
        st_ref[...] = jnp.zeros_like(st_ref)

    s = jnp.sum(out, axis=0)
    s2 = jnp.sum(out * out, axis=0)
    st_ref[0] += s
    st_ref[1] += s2

    @pl.when(i == GRID - 1)
    def _():
        mu = st_ref[0] / N
        st_ref[0] = mu
        st_ref[1] = st_ref[1] / N - mu * mu


def _combine(split, H, ch, num, den, h, aa, b):
    D = H * ch
    Dp = num.shape[2]
    return pl.pallas_call(
        functools.partial(_combine_body, split, H, ch),
        grid=(GRID,),
        in_specs=[
            pl.BlockSpec((2, ROWS, Dp), lambda i: (0, i, 0)),
            pl.BlockSpec((2, ROWS, 16), lambda i: (0, i, 0)),
            pl.BlockSpec((ROWS, D), lambda i: (i, 0)),
            pl.BlockSpec((ROWS, 16), lambda i: (i, 0)),
            pl.BlockSpec((D,), lambda i: (0,)),
        ],
        out_specs=[
            pl.BlockSpec((ROWS, D), lambda i: (i, 0)),
            pl.BlockSpec((2, D), lambda i: (0, 0)),
        ],
        out_shape=[
            jax.ShapeDtypeStruct((N, D), jnp.float32),
            jax.ShapeDtypeStruct((2, D), jnp.float32),
        ],
    )(num, den, h, aa, b)


def _final_body(num_ref, den_ref, h_ref, aa_ref, b_ref, out_ref):
    asv = aa_ref[:, 0:1]
    adv = aa_ref[:, 8:9]
    wself = jnp.exp(_leaky(asv + adv))
    den = den_ref[0, :, 0:1] + den_ref[1, :, 0:1] + wself
    num = num_ref[0] + num_ref[1] + wself * h_ref[...]
    out = num / den + b_ref[...][None, :]
    m = jnp.max(out, axis=1, keepdims=True)
    lse = m + jnp.log(jnp.sum(jnp.exp(out - m), axis=1, keepdims=True))
    out_ref[...] = out - lse


def _final(num, den, h, aa, b):
    return pl.pallas_call(
        _final_body,
        grid=(GRID,),
        in_specs=[
            pl.BlockSpec((2, ROWS, 16), lambda i: (0, i, 0)),
            pl.BlockSpec((2, ROWS, 16), lambda i: (0, i, 0)),
            pl.BlockSpec((ROWS, 16), lambda i: (i, 0)),
            pl.BlockSpec((ROWS, 16), lambda i: (i, 0)),
            pl.BlockSpec((16,), lambda i: (0,)),
        ],
        out_specs=pl.BlockSpec((ROWS, 16), lambda i: (i, 0)),
        out_shape=jax.ShapeDtypeStruct((N, 16), jnp.float32),
    )(num, den, h, aa, b)


def _edge_sc_body(split, Dp, head_of_block, n_chunks,
                  em_ref, h_ref, asn_ref, adn_ref,
                  num_ref, den_ref,
                  ei0, ei1, asb0, asb1, adb0, adb1,
                  hb0, hb1, wb0, wb1,
                  acc, dacc,
                  s_as0, s_as1, s_ad0, s_ad1, s_h0, s_h1,
                  s_sc0, s_sc1, s_sd0, s_sd1):
    EI = (ei0, ei1)
    ASB = (asb0, asb1)
    ADB = (adb0, adb1)
    HB = (hb0, hb1)
    WB = (wb0, wb1)
    S_AS = (s_as0, s_as1)
    S_AD = (s_ad0, s_ad1)
    S_H = (s_h0, s_h1)
    S_SC = (s_sc0, s_sc1)
    S_SD = (s_sd0, s_sd1)
    c = lax.axis_index("c")
    s = lax.axis_index("s")

    zb = hb0.at[pl.ds(0, DCH)]
    zd = wb0.at[pl.ds(0, DCH)]

    @pl.loop(0, DCH)
    def _z(i):
        for j in range(Dp // 16):
            zb[i, pl.ds(j * 16, 16)] = jnp.zeros((16,), jnp.float32)
        zd[i, pl.ds(0, 16)] = jnp.zeros((16,), jnp.float32)

    for t in range(DRAIN // DCH):
        base = pl.multiple_of(s * DRAIN + t * DCH, DCH)
        pltpu.sync_copy(zb, acc.at[pl.ds(base, DCH)])
        pltpu.sync_copy(zd, dacc.at[pl.ds(base, DCH)])
    plsc.subcore_barrier()

    if split == "edge":
        row_base = c * (EROWSP // 2) + s * n_chunks
        h_v, asn_v, adn_v = h_ref, asn_ref, adn_ref
    else:
        row_base = s * n_chunks
        h_v = h_ref.at[c]
        asn_v = asn_ref.at[c]
        adn_v = adn_ref.at[c]

    def load_idx(row, b):
        pltpu.sync_copy(em_ref.at[row], EI[b])

    def gather_descs(b):
        return (
            pltpu.make_async_copy(asn_v.at[EI[b].at[0]], ASB[b], S_AS[b]),
            pltpu.make_async_copy(adn_v.at[EI[b].at[1]], ADB[b], S_AD[b]),
            pltpu.make_async_copy(h_v.at[EI[b].at[0]], HB[b], S_H[b]),
        )

    def start_gathers(b):
        for d in gather_descs(b):
            d.start()

    def wait_gathers(b):
        for d in gather_descs(b):
            d.wait()

    def compute(b):
        asb, adb, hbuf, wbuf = ASB[b], ADB[b], HB[b], WB[b]

        @pl.loop(0, CH)
        def _row(r):
            e = asb[r] + adb[r]
            e = jnp.where(e > 0.0, e, 0.2 * e)
            w = jnp.exp(e)
            wbuf[r] = w
            for j in range(Dp // 16):
                ws = w[head_of_block[j]]
                hbuf[r, pl.ds(j * 16, 16)] = hbuf[r, pl.ds(j * 16, 16)] * ws

    def scatter(b):
        d1 = pltpu.async_copy(HB[b], acc.at[EI[b].at[1]], S_SC[b], add=True)
        d2 = pltpu.async_copy(WB[b], dacc.at[EI[b].at[1]], S_SD[b], add=True)
        d1.wait()
        d2.wait()

    load_idx(row_base, 0)
    start_gathers(0)
    load_idx(row_base + 1, 1)
    start_gathers(1)

    nh = n_chunks // 2

    @pl.loop(0, nh - 1)
    def _pair(kk):
        k0 = row_base + 2 * kk
        for b in (0, 1):
            wait_gathers(b)
            compute(b)
            scatter(b)
            load_idx(k0 + b + 2, b)
            start_gathers(b)

    for b in (0, 1):
        wait_gathers(b)
        compute(b)
        scatter(b)

    plsc.subcore_barrier()

    db = (hb0.at[pl.ds(0, DCH)], hb1.at[pl.ds(0, DCH)])
    dd = (wb0.at[pl.ds(0, DCH)], wb1.at[pl.ds(0, DCH)])
    for t in range(DRAIN // DCH):
        b = t % 2
        base = pl.multiple_of(s * DRAIN + t * DCH, DCH)
        pltpu.sync_copy(acc.at[pl.ds(base, DCH)], db[b])
        pltpu.sync_copy(db[b], num_ref.at[c].at[pl.ds(base, DCH)])
        pltpu.sync_copy(dacc.at[pl.ds(base, DCH)], dd[b])
        pltpu.sync_copy(dd[b], den_ref.at[c].at[pl.ds(base, DCH)])


def _edge_sc(split, H, ch, h, asn, adn, em):
    D = H * ch
    Dp = D if split == "edge" else D // 2
    if split == "edge":
        head_of_block = [min(j * 16 // ch, H - 1) for j in range(Dp // 16)]
        n_chunks = EROWSP // 2 // NSUB
    else:
        head_of_block = [j * 16 // ch for j in range(Dp // 16)]
        n_chunks = EROWSP // NSUB
    body = functools.partial(_edge_sc_body, split, Dp, head_of_block, n_chunks)
    mesh = plsc.VectorSubcoreMesh(core_axis_name="c", subcore_axis_name="s")
    kfn = pl.kernel(
        body,
        out_type=[
            jax.ShapeDtypeStruct((2, NP, Dp), jnp.float32),
            jax.ShapeDtypeStruct((2, NP, 16), jnp.float32),
        ],
        mesh=mesh,
        compiler_params=pltpu.CompilerParams(use_tc_tiling_on_sc=False),
        scratch_types=(
            [pltpu.VMEM((2, CH), jnp.int32)] * 2
            + [pltpu.VMEM((CH, 16), jnp.float32)] * 4
            + [pltpu.VMEM((CH, Dp), jnp.float32)] * 2
            + [pltpu.VMEM((CH, 16), jnp.float32)] * 2
            + [
                pltpu.VMEM_SHARED((NP, Dp), jnp.float32),
                pltpu.VMEM_SHARED((NP, 16), jnp.float32),
            ]
            + [pltpu.SemaphoreType.DMA] * 10
        ),
    )
    return kfn(em, h, asn, adn)


def _edges_jnp(h, asn, adn, src, dst, H, ch, split):
    D = H * ch
    e = asn[src] + adn[dst]
    w = jnp.exp(_leaky(e))
    den = jax.ops.segment_sum(w, dst, num_segments=N)
    msg = h[src].reshape(E, H, ch) * w[..., None]
    num = jax.ops.segment_sum(msg, dst, num_segments=N).reshape(N, D)
    den16 = jnp.zeros((N, 16), jnp.float32)
    if split == "edge":
        den16 = den16.at[:, 0:H].set(den)
        num_p = jnp.stack([num, jnp.zeros_like(num)])
        den_p = jnp.stack([den16, jnp.zeros_like(den16)])
    else:
        Hh = H // 2
        num_p = jnp.stack([num[:, : D // 2], num[:, D // 2:]])
        den_p = jnp.stack([
            den16.at[:, 0:Hh].set(den[:, 0:Hh]),
            den16.at[:, 0:Hh].set(den[:, Hh:]),
        ])
    return num_p, den_p


def _build_ac(a_s, a_d, D):
    H, ch = a_s.shape
    A = jnp.zeros((D, 16), jnp.float32)
    for hh in range(H):
        A = A.at[hh * ch:(hh + 1) * ch, hh].set(a_s[hh])
        A = A.at[hh * ch:(hh + 1) * ch, 8 + hh].set(a_d[hh])
    return A


def kernel(x, edge_index, W1, a1s, a1d, b1, g1, be1, W2, a2s, a2d, b2, g2, be2,
           W3, a3s, a3d, b3):
    src = edge_index[0]
    dst = edge_index[1]
    pad = N + jnp.arange(E_PAD - E, dtype=jnp.int32) % (NP - N)
    srcp = jnp.concatenate([src, pad]).reshape(EROWSP, CH)
    dstp = jnp.concatenate([dst, pad]).reshape(EROWSP, CH)
    em = jnp.stack([srcp, dstp], axis=1)

    h1, aa1 = _dense(x, W1, _build_ac(a1s, a1d, 128))
    asn = jnp.pad(aa1[:, 0:8], ((0, 0), (0, 8)))
    adn = jnp.pad(aa1[:, 8:16], ((0, 0), (0, 8)))
    num, den = _edge_sc("edge", 8, 16, h1, asn, adn, em)
    out1, st1 = _combine("edge", 8, 16, num, den, h1, aa1, b1)

    h2, aa2 = _dense(out1, W2, _build_ac(a2s, a2d, 256), st1, g1, be1)
    h2s = jnp.stack([h2[:, 0:128], h2[:, 128:256]])
    asn2 = jnp.stack([jnp.pad(aa2[:, 0:4], ((0, 0), (0, 12))),
                      jnp.pad(aa2[:, 4:8], ((0, 0), (0, 12)))])
    adn2 = jnp.stack([jnp.pad(aa2[:, 8:12], ((0, 0), (0, 12))),
                      jnp.pad(aa2[:, 12:16], ((0, 0), (0, 12)))])
    num, den = _edge_sc("chan", 8, 32, h2s, asn2, adn2, em)
    out2, st2 = _combine("chan", 8, 32, num, den, h2, aa2, b2)

    h3, aa3 = _dense(out2, W3, _build_ac(a3s, a3d, 16), st2, g2, be2)
    asn = jnp.pad(aa3[:, 0:1], ((0, 0), (0, 15)))
    adn = jnp.pad(aa3[:, 8:9], ((0, 0), (0, 15)))
    num, den = _edge_sc("edge", 1, 16, h3, asn, adn, em)
    return _final(num, den, h3, aa3, b3)


# speedup vs baseline: 1.5518x; 1.1401x over previous
import functools

import jax
import jax.numpy as jnp
from jax import lax
from jax.experimental import pallas as pl
from jax.experimental.pallas import tpu as pltpu
from jax.experimental.pallas import tpu_sc as plsc

N = 10000
ROWS = 1000
GRID = N // ROWS
E = 320000
CH = 96
EROWSP = 3456
E_PAD = EROWSP * CH
NSUB = 16
NP = 10240
DRAIN = NP // NSUB
DCH = 64


def _leaky(e):
    return jnp.where(e > 0, e, 0.2 * e)


def _elu(x):
    return jnp.where(x > 0, x, jnp.exp(x) - 1.0)


def _dense_first_body(x_ref, w_ref, ac_ref, h_ref, aa_ref):
    h = jnp.dot(x_ref[...], w_ref[...], preferred_element_type=jnp.float32)
    h_ref[...] = h
    aa_ref[...] = jnp.dot(h, ac_ref[...], preferred_element_type=jnp.float32)


def _dense_bn_body(x_ref, st_ref, g_ref, be_ref, w_ref, ac_ref, h_ref, aa_ref):
    mu = st_ref[0]
    var = st_ref[1]
    rstd = jax.lax.rsqrt(var + 1e-5)
    xn = (x_ref[...] - mu[None, :]) * (rstd * g_ref[...])[None, :] + be_ref[...][None, :]
    xe = _elu(xn)
    h = jnp.dot(xe, w_ref[...], preferred_element_type=jnp.float32)
    h_ref[...] = h
    aa_ref[...] = jnp.dot(h, ac_ref[...], preferred_element_type=jnp.float32)


def _dense(x, W, Ac, stats=None, g=None, be=None):
    Din = x.shape[1]
    D = W.shape[1]
    if stats is None:
        return pl.pallas_call(
            _dense_first_body,
            grid=(GRID,),
            in_specs=[
                pl.BlockSpec((ROWS, Din), lambda i: (i, 0)),
                pl.BlockSpec((Din, D), lambda i: (0, 0)),
                pl.BlockSpec((D, 16), lambda i: (0, 0)),
            ],
            out_specs=[
                pl.BlockSpec((ROWS, D), lambda i: (i, 0)),
                pl.BlockSpec((ROWS, 16), lambda i: (i, 0)),
            ],
            out_shape=[
                jax.ShapeDtypeStruct((NP, D), jnp.float32),
                jax.ShapeDtypeStruct((NP, 16), jnp.float32),
            ],
        )(x, W, Ac)
    return pl.pallas_call(
        _dense_bn_body,
        grid=(GRID,),
        in_specs=[
            pl.BlockSpec((ROWS, Din), lambda i: (i, 0)),
            pl.BlockSpec((2, Din), lambda i: (0, 0)),
            pl.BlockSpec((Din,), lambda i: (0,)),
            pl.BlockSpec((Din,), lambda i: (0,)),
            pl.BlockSpec((Din, D), lambda i: (0, 0)),
            pl.BlockSpec((D, 16), lambda i: (0, 0)),
        ],
        out_specs=[
            pl.BlockSpec((ROWS, D), lambda i: (i, 0)),
            pl.BlockSpec((ROWS, 16), lambda i: (i, 0)),
        ],
        out_shape=[
            jax.ShapeDtypeStruct((NP, D), jnp.float32),
            jax.ShapeDtypeStruct((NP, 16), jnp.float32),
        ],
    )(x, stats, g, be, W, Ac)


def _combine_body(split, H, ch, num_ref, den_ref, h_ref, aa_ref, b_ref,
                  out_ref, st_ref):
    i = pl.program_id(0)
    D = H * ch
    asv = aa_ref[:, 0:8][:, 0:H]
    adv = aa_ref[:, 8:16][:, 0:H]
    wself = jnp.exp(_leaky(asv + adv))
    h3 = h_ref[...].reshape(ROWS, H, ch)
    if split == "edge":
        num3 = (num_ref[0] + num_ref[1]).reshape(ROWS, H, ch)
        den = den_ref[0, :, 0:H] + den_ref[1, :, 0:H]
    else:
        Hh = H // 2
        num3 = jnp.concatenate(
            [num_ref[0].reshape(ROWS, Hh, ch), num_ref[1].reshape(ROWS, Hh, ch)],
            axis=1)
        den = jnp.concatenate([den_ref[0, :, 0:Hh], den_ref[1, :, 0:Hh]], axis=1)
    den = den + wself
    num3 = num3 + wself[:, :, None] * h3
    out = num3 / den[:, :, None]
    out = out.reshape(ROWS, D) + b_ref[...][None, :]
    out_ref[...] = out

    @pl.when(i == 0)
    def _():
        st_ref[...] = jnp.zeros_like(st_ref)

    s = jnp.sum(out, axis=0)
    s2 = jnp.sum(out * out, axis=0)
    st_ref[0] += s
    st_ref[1] += s2

    @pl.when(i == GRID - 1)
    def _():
        mu = st_ref[0] / N
        st_ref[0] = mu
        st_ref[1] = st_ref[1] / N - mu * mu


def _combine(split, H, ch, num, den, h, aa, b):
    D = H * ch
    Dp = num.shape[2]
    return pl.pallas_call(
        functools.partial(_combine_body, split, H, ch),
        grid=(GRID,),
        in_specs=[
            pl.BlockSpec((2, ROWS, Dp), lambda i: (0, i, 0)),
            pl.BlockSpec((2, ROWS, 16), lambda i: (0, i, 0)),
            pl.BlockSpec((ROWS, D), lambda i: (i, 0)),
            pl.BlockSpec((ROWS, 16), lambda i: (i, 0)),
            pl.BlockSpec((D,), lambda i: (0,)),
        ],
        out_specs=[
            pl.BlockSpec((ROWS, D), lambda i: (i, 0)),
            pl.BlockSpec((2, D), lambda i: (0, 0)),
        ],
        out_shape=[
            jax.ShapeDtypeStruct((N, D), jnp.float32),
            jax.ShapeDtypeStruct((2, D), jnp.float32),
        ],
    )(num, den, h, aa, b)


def _final_body(num_ref, den_ref, h_ref, aa_ref, b_ref, out_ref):
    asv = aa_ref[:, 0:1]
    adv = aa_ref[:, 8:9]
    wself = jnp.exp(_leaky(asv + adv))
    den = den_ref[0, :, 0:1] + den_ref[1, :, 0:1] + wself
    num = num_ref[0] + num_ref[1] + wself * h_ref[...]
    out = num / den + b_ref[...][None, :]
    m = jnp.max(out, axis=1, keepdims=True)
    lse = m + jnp.log(jnp.sum(jnp.exp(out - m), axis=1, keepdims=True))
    out_ref[...] = out - lse


def _final(num, den, h, aa, b):
    return pl.pallas_call(
        _final_body,
        grid=(GRID,),
        in_specs=[
            pl.BlockSpec((2, ROWS, 16), lambda i: (0, i, 0)),
            pl.BlockSpec((2, ROWS, 16), lambda i: (0, i, 0)),
            pl.BlockSpec((ROWS, 16), lambda i: (i, 0)),
            pl.BlockSpec((ROWS, 16), lambda i: (i, 0)),
            pl.BlockSpec((16,), lambda i: (0,)),
        ],
        out_specs=pl.BlockSpec((ROWS, 16), lambda i: (i, 0)),
        out_shape=jax.ShapeDtypeStruct((N, 16), jnp.float32),
    )(num, den, h, aa, b)


def _edge_sc_body(split, Dp, head_of_block, n_chunks,
                  em_ref, h_ref, asn_ref, adn_ref,
                  num_ref, den_ref,
                  ei0, ei1, ei2, ei3, asb0, asb1, adb0, adb1,
                  hb0, hb1, wb0, wb1,
                  acc, dacc,
                  s_i0, s_i1, s_i2, s_i3,
                  s_as0, s_as1, s_ad0, s_ad1, s_h0, s_h1,
                  s_sc0, s_sc1, s_sd0, s_sd1):
    EI = (ei0, ei1, ei2, ei3)
    ASB = (asb0, asb1)
    ADB = (adb0, adb1)
    HB = (hb0, hb1)
    WB = (wb0, wb1)
    S_I = (s_i0, s_i1, s_i2, s_i3)
    S_AS = (s_as0, s_as1)
    S_AD = (s_ad0, s_ad1)
    S_H = (s_h0, s_h1)
    S_SC = (s_sc0, s_sc1)
    S_SD = (s_sd0, s_sd1)
    c = lax.axis_index("c")
    s = lax.axis_index("s")

    zb = hb0.at[pl.ds(0, DCH)]
    zd = wb0.at[pl.ds(0, DCH)]

    @pl.loop(0, DCH)
    def _z(i):
        for j in range(Dp // 16):
            zb[i, pl.ds(j * 16, 16)] = jnp.zeros((16,), jnp.float32)
        zd[i, pl.ds(0, 16)] = jnp.zeros((16,), jnp.float32)

    for t in range(DRAIN // DCH):
        base = pl.multiple_of(s * DRAIN + t * DCH, DCH)
        pltpu.sync_copy(zb, acc.at[pl.ds(base, DCH)])
        pltpu.sync_copy(zd, dacc.at[pl.ds(base, DCH)])
    plsc.subcore_barrier()

    if split == "edge":
        row_base = c * (EROWSP // 2) + s * n_chunks
        h_v, asn_v, adn_v = h_ref, asn_ref, adn_ref
    else:
        row_base = s * n_chunks
        h_v = h_ref.at[c]
        asn_v = asn_ref.at[c]
        adn_v = adn_ref.at[c]

    def idx_desc(row, e):
        return pltpu.make_async_copy(em_ref.at[row], EI[e], S_I[e])

    def gather_descs(b, e):
        return (
            pltpu.make_async_copy(asn_v.at[EI[e].at[0]], ASB[b], S_AS[b]),
            pltpu.make_async_copy(adn_v.at[EI[e].at[1]], ADB[b], S_AD[b]),
            pltpu.make_async_copy(h_v.at[EI[e].at[0]], HB[b], S_H[b]),
        )

    def scatter_descs(b, e):
        return (
            pltpu.make_async_copy(HB[b], acc.at[EI[e].at[1]], S_SC[b]),
            pltpu.make_async_copy(WB[b], dacc.at[EI[e].at[1]], S_SD[b]),
        )

    def start_scatter(b, e):
        pltpu.async_copy(HB[b], acc.at[EI[e].at[1]], S_SC[b], add=True)
        pltpu.async_copy(WB[b], dacc.at[EI[e].at[1]], S_SD[b], add=True)

    def compute(b):
        asb, adb, hbuf, wbuf = ASB[b], ADB[b], HB[b], WB[b]

        @pl.loop(0, CH)
        def _row(r):
            e = asb[r] + adb[r]
            e = jnp.where(e > 0.0, e, 0.2 * e)
            w = jnp.exp(e)
            wbuf[r] = w
            for j in range(Dp // 16):
                ws = w[head_of_block[j]]
                hbuf[r, pl.ds(j * 16, 16)] = hbuf[r, pl.ds(j * 16, 16)] * ws

    def step(k, j, first=False):
        b = j % 2
        bn = (b + 1) % 2
        e = j % 4
        en = (j + 1) % 4
        e2 = (j + 2) % 4
        ep = (j + 3) % 4
        if first:
            idx_desc(row_base + k + 2, e2).start()
        else:
            idx_desc(row_base + jnp.minimum(k + 2, n_chunks - 1), e2).start()
            for d in scatter_descs(bn, ep):
                d.wait()
            idx_desc(row_base + k + 1, en).wait()
        for d in gather_descs(bn, en):
            d.start()
        for d in gather_descs(b, e):
            d.wait()
        compute(b)
        start_scatter(b, e)

    pltpu.sync_copy(em_ref.at[row_base], EI[0])
    pltpu.sync_copy(em_ref.at[row_base + 1], EI[1])
    for d in gather_descs(0, 0):
        d.start()
    step(0, 0, first=True)
    step(1, 1)
    step(2, 2)

    @pl.loop(0, (n_chunks - 4) // 4)
    def _quad(t):
        k0 = 3 + 4 * t
        for j4 in range(4):
            step(k0 + j4, (3 + j4) % 4)

    idx_desc(row_base, 0).wait()
    for d in gather_descs(1, 3):
        d.wait()
    compute(1)
    start_scatter(1, 3)
    for d in scatter_descs(0, 2):
        d.wait()
    for d in scatter_descs(1, 3):
        d.wait()

    plsc.subcore_barrier()

    db = (hb0.at[pl.ds(0, DCH)], hb1.at[pl.ds(0, DCH)])
    dd = (wb0.at[pl.ds(0, DCH)], wb1.at[pl.ds(0, DCH)])
    for t in range(DRAIN // DCH):
        b = t % 2
        base = pl.multiple_of(s * DRAIN + t * DCH, DCH)
        pltpu.sync_copy(acc.at[pl.ds(base, DCH)], db[b])
        pltpu.sync_copy(db[b], num_ref.at[c].at[pl.ds(base, DCH)])
        pltpu.sync_copy(dacc.at[pl.ds(base, DCH)], dd[b])
        pltpu.sync_copy(dd[b], den_ref.at[c].at[pl.ds(base, DCH)])


def _edge_sc(split, H, ch, h, asn, adn, em):
    D = H * ch
    Dp = D if split == "edge" else D // 2
    if split == "edge":
        head_of_block = [min(j * 16 // ch, H - 1) for j in range(Dp // 16)]
        n_chunks = EROWSP // 2 // NSUB
    else:
        head_of_block = [j * 16 // ch for j in range(Dp // 16)]
        n_chunks = EROWSP // NSUB
    body = functools.partial(_edge_sc_body, split, Dp, head_of_block, n_chunks)
    mesh = plsc.VectorSubcoreMesh(core_axis_name="c", subcore_axis_name="s")
    kfn = pl.kernel(
        body,
        out_type=[
            jax.ShapeDtypeStruct((2, NP, Dp), jnp.float32),
            jax.ShapeDtypeStruct((2, NP, 16), jnp.float32),
        ],
        mesh=mesh,
        compiler_params=pltpu.CompilerParams(use_tc_tiling_on_sc=False),
        scratch_types=(
            [pltpu.VMEM((2, CH), jnp.int32)] * 4
            + [pltpu.VMEM((CH, 16), jnp.float32)] * 4
            + [pltpu.VMEM((CH, Dp), jnp.float32)] * 2
            + [pltpu.VMEM((CH, 16), jnp.float32)] * 2
            + [
                pltpu.VMEM_SHARED((NP, Dp), jnp.float32),
                pltpu.VMEM_SHARED((NP, 16), jnp.float32),
            ]
            + [pltpu.SemaphoreType.DMA] * 14
        ),
    )
    return kfn(em, h, asn, adn)


def _edges_jnp(h, asn, adn, src, dst, H, ch, split):
    D = H * ch
    e = asn[src] + adn[dst]
    w = jnp.exp(_leaky(e))
    den = jax.ops.segment_sum(w, dst, num_segments=N)
    msg = h[src].reshape(E, H, ch) * w[..., None]
    num = jax.ops.segment_sum(msg, dst, num_segments=N).reshape(N, D)
    den16 = jnp.zeros((N, 16), jnp.float32)
    if split == "edge":
        den16 = den16.at[:, 0:H].set(den)
        num_p = jnp.stack([num, jnp.zeros_like(num)])
        den_p = jnp.stack([den16, jnp.zeros_like(den16)])
    else:
        Hh = H // 2
        num_p = jnp.stack([num[:, : D // 2], num[:, D // 2:]])
        den_p = jnp.stack([
            den16.at[:, 0:Hh].set(den[:, 0:Hh]),
            den16.at[:, 0:Hh].set(den[:, Hh:]),
        ])
    return num_p, den_p


def _build_ac(a_s, a_d, D):
    H, ch = a_s.shape
    A = jnp.zeros((D, 16), jnp.float32)
    for hh in range(H):
        A = A.at[hh * ch:(hh + 1) * ch, hh].set(a_s[hh])
        A = A.at[hh * ch:(hh + 1) * ch, 8 + hh].set(a_d[hh])
    return A


def kernel(x, edge_index, W1, a1s, a1d, b1, g1, be1, W2, a2s, a2d, b2, g2, be2,
           W3, a3s, a3d, b3):
    src = edge_index[0]
    dst = edge_index[1]
    pad = N + jnp.arange(E_PAD - E, dtype=jnp.int32) % (NP - N)
    srcp = jnp.concatenate([src, pad]).reshape(EROWSP, CH)
    dstp = jnp.concatenate([dst, pad]).reshape(EROWSP, CH)
    em = jnp.stack([srcp, dstp], axis=1)

    h1, aa1 = _dense(x, W1, _build_ac(a1s, a1d, 128))
    asn = jnp.pad(aa1[:, 0:8], ((0, 0), (0, 8)))
    adn = jnp.pad(aa1[:, 8:16], ((0, 0), (0, 8)))
    num, den = _edge_sc("edge", 8, 16, h1, asn, adn, em)
    out1, st1 = _combine("edge", 8, 16, num, den, h1, aa1, b1)

    h2, aa2 = _dense(out1, W2, _build_ac(a2s, a2d, 256), st1, g1, be1)
    h2s = jnp.stack([h2[:, 0:128], h2[:, 128:256]])
    asn2 = jnp.stack([jnp.pad(aa2[:, 0:4], ((0, 0), (0, 12))),
                      jnp.pad(aa2[:, 4:8], ((0, 0), (0, 12)))])
    adn2 = jnp.stack([jnp.pad(aa2[:, 8:12], ((0, 0), (0, 12))),
                      jnp.pad(aa2[:, 12:16], ((0, 0), (0, 12)))])
    num, den = _edge_sc("chan", 8, 32, h2s, asn2, adn2, em)
    out2, st2 = _combine("chan", 8, 32, num, den, h2, aa2, b2)

    h3, aa3 = _dense(out2, W3, _build_ac(a3s, a3d, 16), st2, g2, be2)
    asn = jnp.pad(aa3[:, 0:1], ((0, 0), (0, 15)))
    adn = jnp.pad(aa3[:, 8:9], ((0, 0), (0, 15)))
    num, den = _edge_sc("edge", 1, 16, h3, asn, adn, em)
    return _final(num, den, h3, aa3, b3)


# speedup vs baseline: 1.5641x; 1.0079x over previous
import functools

import jax
import jax.numpy as jnp
from jax import lax
from jax.experimental import pallas as pl
from jax.experimental.pallas import tpu as pltpu
from jax.experimental.pallas import tpu_sc as plsc

N = 10000
ROWS = 1000
GRID = N // ROWS
E = 320000
CH = 96
EROWSP = 3456
E_PAD = EROWSP * CH
NSUB = 16
NP = 10240
DRAIN = NP // NSUB
DCH = 64


def _leaky(e):
    return jnp.where(e > 0, e, 0.2 * e)


def _elu(x):
    return jnp.where(x > 0, x, jnp.exp(x) - 1.0)


def _dense_first_body(x_ref, w_ref, ac_ref, h_ref, aa_ref):
    h = jnp.dot(x_ref[...], w_ref[...], preferred_element_type=jnp.float32)
    h_ref[...] = h
    aa_ref[...] = jnp.dot(h, ac_ref[...], preferred_element_type=jnp.float32)


def _dense_bn_body(x_ref, st_ref, g_ref, be_ref, w_ref, ac_ref, h_ref, aa_ref):
    mu = st_ref[0]
    var = st_ref[1]
    rstd = jax.lax.rsqrt(var + 1e-5)
    xn = (x_ref[...] - mu[None, :]) * (rstd * g_ref[...])[None, :] + be_ref[...][None, :]
    xe = _elu(xn)
    h = jnp.dot(xe, w_ref[...], preferred_element_type=jnp.float32)
    h_ref[...] = h
    aa_ref[...] = jnp.dot(h, ac_ref[...], preferred_element_type=jnp.float32)


def _dense(x, W, Ac, stats=None, g=None, be=None):
    Din = x.shape[1]
    D = W.shape[1]
    if stats is None:
        return pl.pallas_call(
            _dense_first_body,
            grid=(GRID,),
            in_specs=[
                pl.BlockSpec((ROWS, Din), lambda i: (i, 0)),
                pl.BlockSpec((Din, D), lambda i: (0, 0)),
                pl.BlockSpec((D, 16), lambda i: (0, 0)),
            ],
            out_specs=[
                pl.BlockSpec((ROWS, D), lambda i: (i, 0)),
                pl.BlockSpec((ROWS, 16), lambda i: (i, 0)),
            ],
            out_shape=[
                jax.ShapeDtypeStruct((NP, D), jnp.float32),
                jax.ShapeDtypeStruct((NP, 16), jnp.float32),
            ],
        )(x, W, Ac)
    return pl.pallas_call(
        _dense_bn_body,
        grid=(GRID,),
        in_specs=[
            pl.BlockSpec((ROWS, Din), lambda i: (i, 0)),
            pl.BlockSpec((2, Din), lambda i: (0, 0)),
            pl.BlockSpec((Din,), lambda i: (0,)),
            pl.BlockSpec((Din,), lambda i: (0,)),
            pl.BlockSpec((Din, D), lambda i: (0, 0)),
            pl.BlockSpec((D, 16), lambda i: (0, 0)),
        ],
        out_specs=[
            pl.BlockSpec((ROWS, D), lambda i: (i, 0)),
            pl.BlockSpec((ROWS, 16), lambda i: (i, 0)),
        ],
        out_shape=[
            jax.ShapeDtypeStruct((NP, D), jnp.float32),
            jax.ShapeDtypeStruct((NP, 16), jnp.float32),
        ],
    )(x, stats, g, be, W, Ac)


def _combine_body(split, H, ch, num_ref, den_ref, h_ref, aa_ref, b_ref,
                  out_ref, st_ref):
    i = pl.program_id(0)
    D = H * ch
    asv = aa_ref[:, 0:8][:, 0:H]
    adv = aa_ref[:, 8:16][:, 0:H]
    wself = jnp.exp(_leaky(asv + adv))
    h3 = h_ref[...].reshape(ROWS, H, ch)
    if split == "edge":
        num3 = (num_ref[0] + num_ref[1]).reshape(ROWS, H, ch)
        den = den_ref[0, :, 0:H] + den_ref[1, :, 0:H]
    else:
        Hh = H // 2
        num3 = jnp.concatenate(
            [num_ref[0].reshape(ROWS, Hh, ch), num_ref[1].reshape(ROWS, Hh, ch)],
            axis=1)
        den = jnp.concatenate([den_ref[0, :, 0:Hh], den_ref[1, :, 0:Hh]], axis=1)
    den = den + wself
    num3 = num3 + wself[:, :, None] * h3
    out = num3 / den[:, :, None]
    out = out.reshape(ROWS, D) + b_ref[...][None, :]
    out_ref[...] = out

    @pl.when(i == 0)
    def _():
        st_ref[...] = jnp.zeros_like(st_ref)

    s = jnp.sum(out, axis=0)
    s2 = jnp.sum(out * out, axis=0)
    st_ref[0] += s
    st_ref[1] += s2

    @pl.when(i == GRID - 1)
    def _():
        mu = st_ref[0] / N
        st_ref[0] = mu
        st_ref[1] = st_ref[1] / N - mu * mu


def _combine(split, H, ch, num, den, h, aa, b):
    D = H * ch
    Dp = num.shape[2]
    return pl.pallas_call(
        functools.partial(_combine_body, split, H, ch),
        grid=(GRID,),
        in_specs=[
            pl.BlockSpec((2, ROWS, Dp), lambda i: (0, i, 0)),
            pl.BlockSpec((2, ROWS, 16), lambda i: (0, i, 0)),
            pl.BlockSpec((ROWS, D), lambda i: (i, 0)),
            pl.BlockSpec((ROWS, 16), lambda i: (i, 0)),
            pl.BlockSpec((D,), lambda i: (0,)),
        ],
        out_specs=[
            pl.BlockSpec((ROWS, D), lambda i: (i, 0)),
            pl.BlockSpec((2, D), lambda i: (0, 0)),
        ],
        out_shape=[
            jax.ShapeDtypeStruct((N, D), jnp.float32),
            jax.ShapeDtypeStruct((2, D), jnp.float32),
        ],
    )(num, den, h, aa, b)


def _final_body(num_ref, den_ref, h_ref, aa_ref, b_ref, out_ref):
    asv = aa_ref[:, 0:1]
    adv = aa_ref[:, 8:9]
    wself = jnp.exp(_leaky(asv + adv))
    den = den_ref[0, :, 0:1] + den_ref[1, :, 0:1] + wself
    num = num_ref[0] + num_ref[1] + wself * h_ref[...]
    out = num / den + b_ref[...][None, :]
    m = jnp.max(out, axis=1, keepdims=True)
    lse = m + jnp.log(jnp.sum(jnp.exp(out - m), axis=1, keepdims=True))
    out_ref[...] = out - lse


def _final(num, den, h, aa, b):
    return pl.pallas_call(
        _final_body,
        grid=(GRID,),
        in_specs=[
            pl.BlockSpec((2, ROWS, 16), lambda i: (0, i, 0)),
            pl.BlockSpec((2, ROWS, 16), lambda i: (0, i, 0)),
            pl.BlockSpec((ROWS, 16), lambda i: (i, 0)),
            pl.BlockSpec((ROWS, 16), lambda i: (i, 0)),
            pl.BlockSpec((16,), lambda i: (0,)),
        ],
        out_specs=pl.BlockSpec((ROWS, 16), lambda i: (i, 0)),
        out_shape=jax.ShapeDtypeStruct((N, 16), jnp.float32),
    )(num, den, h, aa, b)


def _edge_sc_body(split, Dp, head_of_block, n_chunks,
                  em_ref, h_ref, asn_ref, adn_ref,
                  num_ref, den_ref,
                  ei0, ei1, ei2, ei3, asb0, asb1, adb0, adb1,
                  hb0, hb1, wb0, wb1,
                  acc, dacc,
                  s_i0, s_i1, s_i2, s_i3,
                  s_as0, s_as1, s_ad0, s_ad1, s_h0, s_h1,
                  s_sc0, s_sc1, s_sd0, s_sd1):
    EI = (ei0, ei1, ei2, ei3)
    ASB = (asb0, asb1)
    ADB = (adb0, adb1)
    HB = (hb0, hb1)
    WB = (wb0, wb1)
    S_I = (s_i0, s_i1, s_i2, s_i3)
    S_AS = (s_as0, s_as1)
    S_AD = (s_ad0, s_ad1)
    S_H = (s_h0, s_h1)
    S_SC = (s_sc0, s_sc1)
    S_SD = (s_sd0, s_sd1)
    c = lax.axis_index("c")
    s = lax.axis_index("s")

    zb = hb0.at[pl.ds(0, DCH)]
    zd = wb0.at[pl.ds(0, DCH)]

    @pl.loop(0, DCH)
    def _z(i):
        for j in range(Dp // 16):
            zb[i, pl.ds(j * 16, 16)] = jnp.zeros((16,), jnp.float32)
        zd[i, pl.ds(0, 16)] = jnp.zeros((16,), jnp.float32)

    for t in range(DRAIN // DCH):
        base = pl.multiple_of(s * DRAIN + t * DCH, DCH)
        pltpu.sync_copy(zb, acc.at[pl.ds(base, DCH)])
        pltpu.sync_copy(zd, dacc.at[pl.ds(base, DCH)])
    plsc.subcore_barrier()

    if split == "edge":
        row_base = c * (EROWSP // 2) + s * n_chunks
        h_v, asn_v, adn_v = h_ref, asn_ref, adn_ref
    else:
        row_base = s * n_chunks
        h_v = h_ref.at[c]
        asn_v = asn_ref.at[c]
        adn_v = adn_ref.at[c]

    def idx_desc(row, e):
        return pltpu.make_async_copy(em_ref.at[row], EI[e], S_I[e])

    def gather_descs(b, e):
        return (
            pltpu.make_async_copy(asn_v.at[EI[e].at[0]], ASB[b], S_AS[b]),
            pltpu.make_async_copy(adn_v.at[EI[e].at[1]], ADB[b], S_AD[b]),
            pltpu.make_async_copy(h_v.at[EI[e].at[0]], HB[b], S_H[b]),
        )

    def scatter_descs(b, e):
        return (
            pltpu.make_async_copy(HB[b], acc.at[EI[e].at[1]], S_SC[b]),
            pltpu.make_async_copy(WB[b], dacc.at[EI[e].at[1]], S_SD[b]),
        )

    def start_scatter(b, e):
        pltpu.async_copy(HB[b], acc.at[EI[e].at[1]], S_SC[b], add=True)
        pltpu.async_copy(WB[b], dacc.at[EI[e].at[1]], S_SD[b], add=True)

    def compute(b):
        asb, adb, hbuf, wbuf = ASB[b], ADB[b], HB[b], WB[b]

        @pl.loop(0, CH, step=2)
        def _row(r0):
            for u in range(2):
                r = r0 + u
                e = asb[r] + adb[r]
                e = jnp.where(e > 0.0, e, 0.2 * e)
                w = jnp.exp(e)
                wbuf[r] = w
                for j in range(Dp // 16):
                    ws = w[head_of_block[j]]
                    hbuf[r, pl.ds(j * 16, 16)] = hbuf[r, pl.ds(j * 16, 16)] * ws

    def step(k, j, first=False):
        b = j % 2
        bn = (b + 1) % 2
        e = j % 4
        en = (j + 1) % 4
        e2 = (j + 2) % 4
        ep = (j + 3) % 4
        if first:
            idx_desc(row_base + k + 2, e2).start()
        else:
            idx_desc(row_base + jnp.minimum(k + 2, n_chunks - 1), e2).start()
            for d in scatter_descs(bn, ep):
                d.wait()
            idx_desc(row_base + k + 1, en).wait()
        for d in gather_descs(bn, en):
            d.start()
        for d in gather_descs(b, e):
            d.wait()
        compute(b)
        start_scatter(b, e)

    pltpu.sync_copy(em_ref.at[row_base], EI[0])
    pltpu.sync_copy(em_ref.at[row_base + 1], EI[1])
    for d in gather_descs(0, 0):
        d.start()
    step(0, 0, first=True)
    step(1, 1)
    step(2, 2)

    @pl.loop(0, (n_chunks - 4) // 4)
    def _quad(t):
        k0 = 3 + 4 * t
        for j4 in range(4):
            step(k0 + j4, (3 + j4) % 4)

    idx_desc(row_base, 0).wait()
    for d in gather_descs(1, 3):
        d.wait()
    compute(1)
    start_scatter(1, 3)
    for d in scatter_descs(0, 2):
        d.wait()
    for d in scatter_descs(1, 3):
        d.wait()

    plsc.subcore_barrier()

    db = (hb0.at[pl.ds(0, DCH)], hb1.at[pl.ds(0, DCH)])
    dd = (wb0.at[pl.ds(0, DCH)], wb1.at[pl.ds(0, DCH)])
    for t in range(DRAIN // DCH):
        b = t % 2
        base = pl.multiple_of(s * DRAIN + t * DCH, DCH)
        pltpu.sync_copy(acc.at[pl.ds(base, DCH)], db[b])
        pltpu.sync_copy(db[b], num_ref.at[c].at[pl.ds(base, DCH)])
        pltpu.sync_copy(dacc.at[pl.ds(base, DCH)], dd[b])
        pltpu.sync_copy(dd[b], den_ref.at[c].at[pl.ds(base, DCH)])


def _edge_sc(split, H, ch, h, asn, adn, em):
    D = H * ch
    Dp = D if split == "edge" else D // 2
    if split == "edge":
        head_of_block = [min(j * 16 // ch, H - 1) for j in range(Dp // 16)]
        n_chunks = EROWSP // 2 // NSUB
    else:
        head_of_block = [j * 16 // ch for j in range(Dp // 16)]
        n_chunks = EROWSP // NSUB
    body = functools.partial(_edge_sc_body, split, Dp, head_of_block, n_chunks)
    mesh = plsc.VectorSubcoreMesh(core_axis_name="c", subcore_axis_name="s")
    kfn = pl.kernel(
        body,
        out_type=[
            jax.ShapeDtypeStruct((2, NP, Dp), jnp.float32),
            jax.ShapeDtypeStruct((2, NP, 16), jnp.float32),
        ],
        mesh=mesh,
        compiler_params=pltpu.CompilerParams(use_tc_tiling_on_sc=False),
        scratch_types=(
            [pltpu.VMEM((2, CH), jnp.int32)] * 4
            + [pltpu.VMEM((CH, 16), jnp.float32)] * 4
            + [pltpu.VMEM((CH, Dp), jnp.float32)] * 2
            + [pltpu.VMEM((CH, 16), jnp.float32)] * 2
            + [
                pltpu.VMEM_SHARED((NP, Dp), jnp.float32),
                pltpu.VMEM_SHARED((NP, 16), jnp.float32),
            ]
            + [pltpu.SemaphoreType.DMA] * 14
        ),
    )
    return kfn(em, h, asn, adn)


def _edges_jnp(h, asn, adn, src, dst, H, ch, split):
    D = H * ch
    e = asn[src] + adn[dst]
    w = jnp.exp(_leaky(e))
    den = jax.ops.segment_sum(w, dst, num_segments=N)
    msg = h[src].reshape(E, H, ch) * w[..., None]
    num = jax.ops.segment_sum(msg, dst, num_segments=N).reshape(N, D)
    den16 = jnp.zeros((N, 16), jnp.float32)
    if split == "edge":
        den16 = den16.at[:, 0:H].set(den)
        num_p = jnp.stack([num, jnp.zeros_like(num)])
        den_p = jnp.stack([den16, jnp.zeros_like(den16)])
    else:
        Hh = H // 2
        num_p = jnp.stack([num[:, : D // 2], num[:, D // 2:]])
        den_p = jnp.stack([
            den16.at[:, 0:Hh].set(den[:, 0:Hh]),
            den16.at[:, 0:Hh].set(den[:, Hh:]),
        ])
    return num_p, den_p


def _build_ac(a_s, a_d, D):
    H, ch = a_s.shape
    A = jnp.zeros((D, 16), jnp.float32)
    for hh in range(H):
        A = A.at[hh * ch:(hh + 1) * ch, hh].set(a_s[hh])
        A = A.at[hh * ch:(hh + 1) * ch, 8 + hh].set(a_d[hh])
    return A


def kernel(x, edge_index, W1, a1s, a1d, b1, g1, be1, W2, a2s, a2d, b2, g2, be2,
           W3, a3s, a3d, b3):
    src = edge_index[0]
    dst = edge_index[1]
    pad = N + jnp.arange(E_PAD - E, dtype=jnp.int32) % (NP - N)
    srcp = jnp.concatenate([src, pad]).reshape(EROWSP, CH)
    dstp = jnp.concatenate([dst, pad]).reshape(EROWSP, CH)
    em = jnp.stack([srcp, dstp], axis=1)

    h1, aa1 = _dense(x, W1, _build_ac(a1s, a1d, 128))
    asn = jnp.pad(aa1[:, 0:8], ((0, 0), (0, 8)))
    adn = jnp.pad(aa1[:, 8:16], ((0, 0), (0, 8)))
    num, den = _edge_sc("edge", 8, 16, h1, asn, adn, em)
    out1, st1 = _combine("edge", 8, 16, num, den, h1, aa1, b1)

    h2, aa2 = _dense(out1, W2, _build_ac(a2s, a2d, 256), st1, g1, be1)
    h2s = jnp.stack([h2[:, 0:128], h2[:, 128:256]])
    asn2 = jnp.stack([jnp.pad(aa2[:, 0:4], ((0, 0), (0, 12))),
                      jnp.pad(aa2[:, 4:8], ((0, 0), (0, 12)))])
    adn2 = jnp.stack([jnp.pad(aa2[:, 8:12], ((0, 0), (0, 12))),
                      jnp.pad(aa2[:, 12:16], ((0, 0), (0, 12)))])
    num, den = _edge_sc("chan", 8, 32, h2s, asn2, adn2, em)
    out2, st2 = _combine("chan", 8, 32, num, den, h2, aa2, b2)

    h3, aa3 = _dense(out2, W3, _build_ac(a3s, a3d, 16), st2, g2, be2)
    asn = jnp.pad(aa3[:, 0:1], ((0, 0), (0, 15)))
    adn = jnp.pad(aa3[:, 8:9], ((0, 0), (0, 15)))
    num, den = _edge_sc("edge", 1, 16, h3, asn, adn, em)
    return _final(num, den, h3, aa3, b3)


# speedup vs baseline: 1.5645x; 1.0003x over previous
import functools

import jax
import jax.numpy as jnp
from jax import lax
from jax.experimental import pallas as pl
from jax.experimental.pallas import tpu as pltpu
from jax.experimental.pallas import tpu_sc as plsc

N = 10000
ROWS = 1000
GRID = N // ROWS
E = 320000
CH = 96
EROWSP = 3456
E_PAD = EROWSP * CH
NSUB = 16
NP = 10240
DRAIN = NP // NSUB
DCH = 64


def _leaky(e):
    return jnp.where(e > 0, e, 0.2 * e)


def _elu(x):
    return jnp.where(x > 0, x, jnp.exp(x) - 1.0)


def _dense_first_body(x_ref, w_ref, ac_ref, h_ref, aa_ref):
    h = jnp.dot(x_ref[...], w_ref[...], preferred_element_type=jnp.float32)
    h_ref[...] = h
    aa_ref[...] = jnp.dot(h, ac_ref[...], preferred_element_type=jnp.float32)


def _dense_bn_body(x_ref, st_ref, g_ref, be_ref, w_ref, ac_ref, h_ref, aa_ref):
    mu = st_ref[0]
    var = st_ref[1]
    rstd = jax.lax.rsqrt(var + 1e-5)
    xn = (x_ref[...] - mu[None, :]) * (rstd * g_ref[...])[None, :] + be_ref[...][None, :]
    xe = _elu(xn)
    h = jnp.dot(xe, w_ref[...], preferred_element_type=jnp.float32)
    h_ref[...] = h
    aa_ref[...] = jnp.dot(h, ac_ref[...], preferred_element_type=jnp.float32)


def _dense(x, W, Ac, stats=None, g=None, be=None):
    Din = x.shape[1]
    D = W.shape[1]
    if stats is None:
        return pl.pallas_call(
            _dense_first_body,
            grid=(GRID,),
            in_specs=[
                pl.BlockSpec((ROWS, Din), lambda i: (i, 0)),
                pl.BlockSpec((Din, D), lambda i: (0, 0)),
                pl.BlockSpec((D, 16), lambda i: (0, 0)),
            ],
            out_specs=[
                pl.BlockSpec((ROWS, D), lambda i: (i, 0)),
                pl.BlockSpec((ROWS, 16), lambda i: (i, 0)),
            ],
            out_shape=[
                jax.ShapeDtypeStruct((NP, D), jnp.float32),
                jax.ShapeDtypeStruct((NP, 16), jnp.float32),
            ],
        )(x, W, Ac)
    return pl.pallas_call(
        _dense_bn_body,
        grid=(GRID,),
        in_specs=[
            pl.BlockSpec((ROWS, Din), lambda i: (i, 0)),
            pl.BlockSpec((2, Din), lambda i: (0, 0)),
            pl.BlockSpec((Din,), lambda i: (0,)),
            pl.BlockSpec((Din,), lambda i: (0,)),
            pl.BlockSpec((Din, D), lambda i: (0, 0)),
            pl.BlockSpec((D, 16), lambda i: (0, 0)),
        ],
        out_specs=[
            pl.BlockSpec((ROWS, D), lambda i: (i, 0)),
            pl.BlockSpec((ROWS, 16), lambda i: (i, 0)),
        ],
        out_shape=[
            jax.ShapeDtypeStruct((NP, D), jnp.float32),
            jax.ShapeDtypeStruct((NP, 16), jnp.float32),
        ],
    )(x, stats, g, be, W, Ac)


def _combine_body(split, H, ch, num_ref, den_ref, h_ref, aa_ref, b_ref,
                  out_ref, st_ref):
    i = pl.program_id(0)
    D = H * ch
    asv = aa_ref[:, 0:8][:, 0:H]
    adv = aa_ref[:, 8:16][:, 0:H]
    wself = jnp.exp(_leaky(asv + adv))
    h3 = h_ref[...].reshape(ROWS, H, ch)
    if split == "edge":
        num3 = (num_ref[0] + num_ref[1]).reshape(ROWS, H, ch)
        den = den_ref[0, :, 0:H] + den_ref[1, :, 0:H]
    else:
        Hh = H // 2
        num3 = jnp.concatenate(
            [num_ref[0].reshape(ROWS, Hh, ch), num_ref[1].reshape(ROWS, Hh, ch)],
            axis=1)
        den = jnp.concatenate([den_ref[0, :, 0:Hh], den_ref[1, :, 0:Hh]], axis=1)
    den = den + wself
    num3 = num3 + wself[:, :, None] * h3
    out = num3 / den[:, :, None]
    out = out.reshape(ROWS, D) + b_ref[...][None, :]
    out_ref[...] = out

    @pl.when(i == 0)
    def _():
        st_ref[...] = jnp.zeros_like(st_ref)

    s = jnp.sum(out, axis=0)
    s2 = jnp.sum(out * out, axis=0)
    st_ref[0] += s
    st_ref[1] += s2

    @pl.when(i == GRID - 1)
    def _():
        mu = st_ref[0] / N
        st_ref[0] = mu
        st_ref[1] = st_ref[1] / N - mu * mu


def _combine(split, H, ch, num, den, h, aa, b):
    D = H * ch
    Dp = num.shape[2]
    return pl.pallas_call(
        functools.partial(_combine_body, split, H, ch),
        grid=(GRID,),
        in_specs=[
            pl.BlockSpec((2, ROWS, Dp), lambda i: (0, i, 0)),
            pl.BlockSpec((2, ROWS, 16), lambda i: (0, i, 0)),
            pl.BlockSpec((ROWS, D), lambda i: (i, 0)),
            pl.BlockSpec((ROWS, 16), lambda i: (i, 0)),
            pl.BlockSpec((D,), lambda i: (0,)),
        ],
        out_specs=[
            pl.BlockSpec((ROWS, D), lambda i: (i, 0)),
            pl.BlockSpec((2, D), lambda i: (0, 0)),
        ],
        out_shape=[
            jax.ShapeDtypeStruct((N, D), jnp.float32),
            jax.ShapeDtypeStruct((2, D), jnp.float32),
        ],
    )(num, den, h, aa, b)


def _final_body(num_ref, den_ref, h_ref, aa_ref, b_ref, out_ref):
    asv = aa_ref[:, 0:1]
    adv = aa_ref[:, 8:9]
    wself = jnp.exp(_leaky(asv + adv))
    den = den_ref[0, :, 0:1] + den_ref[1, :, 0:1] + wself
    num = num_ref[0] + num_ref[1] + wself * h_ref[...]
    out = num / den + b_ref[...][None, :]
    m = jnp.max(out, axis=1, keepdims=True)
    lse = m + jnp.log(jnp.sum(jnp.exp(out - m), axis=1, keepdims=True))
    out_ref[...] = out - lse


def _final(num, den, h, aa, b):
    return pl.pallas_call(
        _final_body,
        grid=(GRID,),
        in_specs=[
            pl.BlockSpec((2, ROWS, 16), lambda i: (0, i, 0)),
            pl.BlockSpec((2, ROWS, 16), lambda i: (0, i, 0)),
            pl.BlockSpec((ROWS, 16), lambda i: (i, 0)),
            pl.BlockSpec((ROWS, 16), lambda i: (i, 0)),
            pl.BlockSpec((16,), lambda i: (0,)),
        ],
        out_specs=pl.BlockSpec((ROWS, 16), lambda i: (i, 0)),
        out_shape=jax.ShapeDtypeStruct((N, 16), jnp.float32),
    )(num, den, h, aa, b)


def _edge_sc_body(split, Dp, head_of_block, n_chunks,
                  em_ref, h_ref, asn_ref, adn_ref,
                  num_ref, den_ref,
                  ei0, ei1, ei2, ei3, asb0, asb1, adb0, adb1,
                  hb0, hb1, wb0, wb1,
                  acc, dacc,
                  s_i0, s_i1, s_i2, s_i3,
                  s_as0, s_as1, s_ad0, s_ad1, s_h0, s_h1,
                  s_sc0, s_sc1, s_sd0, s_sd1):
    EI = (ei0, ei1, ei2, ei3)
    ASB = (asb0, asb1)
    ADB = (adb0, adb1)
    HB = (hb0, hb1)
    WB = (wb0, wb1)
    S_I = (s_i0, s_i1, s_i2, s_i3)
    S_AS = (s_as0, s_as1)
    S_AD = (s_ad0, s_ad1)
    S_H = (s_h0, s_h1)
    S_SC = (s_sc0, s_sc1)
    S_SD = (s_sd0, s_sd1)
    c = lax.axis_index("c")
    s = lax.axis_index("s")

    zb = hb0.at[pl.ds(0, DCH)]
    zd = wb0.at[pl.ds(0, DCH)]

    @pl.loop(0, DCH)
    def _z(i):
        for j in range(Dp // 16):
            zb[i, pl.ds(j * 16, 16)] = jnp.zeros((16,), jnp.float32)
        zd[i, pl.ds(0, 16)] = jnp.zeros((16,), jnp.float32)

    for t in range(DRAIN // DCH):
        base = pl.multiple_of(s * DRAIN + t * DCH, DCH)
        pltpu.sync_copy(zb, acc.at[pl.ds(base, DCH)])
        pltpu.sync_copy(zd, dacc.at[pl.ds(base, DCH)])
    plsc.subcore_barrier()

    if split == "edge":
        row_base = c * (EROWSP // 2) + s * n_chunks
        h_v, asn_v, adn_v = h_ref, asn_ref, adn_ref
    else:
        row_base = s * n_chunks
        h_v = h_ref.at[c]
        asn_v = asn_ref.at[c]
        adn_v = adn_ref.at[c]

    def idx_desc(row, e):
        return pltpu.make_async_copy(em_ref.at[row], EI[e], S_I[e])

    def gather_descs(b, e):
        return (
            pltpu.make_async_copy(asn_v.at[EI[e].at[0]], ASB[b], S_AS[b]),
            pltpu.make_async_copy(adn_v.at[EI[e].at[1]], ADB[b], S_AD[b]),
            pltpu.make_async_copy(h_v.at[EI[e].at[0]], HB[b], S_H[b]),
        )

    def scatter_descs(b, e):
        return (
            pltpu.make_async_copy(HB[b], acc.at[EI[e].at[1]], S_SC[b]),
            pltpu.make_async_copy(WB[b], dacc.at[EI[e].at[1]], S_SD[b]),
        )

    def start_scatter(b, e):
        pltpu.async_copy(HB[b], acc.at[EI[e].at[1]], S_SC[b], add=True)
        pltpu.async_copy(WB[b], dacc.at[EI[e].at[1]], S_SD[b], add=True)

    def compute(b):
        asb, adb, hbuf, wbuf = ASB[b], ADB[b], HB[b], WB[b]

        @pl.loop(0, CH, step=2)
        def _row(r0):
            for u in range(2):
                r = r0 + u
                e = asb[r] + adb[r]
                e = jnp.where(e > 0.0, e, 0.2 * e)
                w = jnp.exp(e)
                wbuf[r] = w
                for j in range(Dp // 16):
                    ws = w[head_of_block[j]]
                    hbuf[r, pl.ds(j * 16, 16)] = hbuf[r, pl.ds(j * 16, 16)] * ws

    def step(k, j, first=False):
        b = j % 2
        bn = (b + 1) % 2
        e = j % 4
        en = (j + 1) % 4
        e2 = (j + 2) % 4
        ep = (j + 3) % 4
        if first:
            idx_desc(row_base + k + 2, e2).start()
        else:
            idx_desc(row_base + jnp.minimum(k + 2, n_chunks - 1), e2).start()
            for d in scatter_descs(bn, ep):
                d.wait()
            idx_desc(row_base + k + 1, en).wait()
        for d in gather_descs(bn, en):
            d.start()
        for d in gather_descs(b, e):
            d.wait()
        compute(b)
        start_scatter(b, e)

    pltpu.sync_copy(em_ref.at[row_base], EI[0])
    pltpu.sync_copy(em_ref.at[row_base + 1], EI[1])
    for d in gather_descs(0, 0):
        d.start()
    step(0, 0, first=True)
    step(1, 1)
    step(2, 2)

    @pl.loop(0, (n_chunks - 4) // 4)
    def _quad(t):
        k0 = 3 + 4 * t
        for j4 in range(4):
            step(k0 + j4, (3 + j4) % 4)

    idx_desc(row_base, 0).wait()
    for d in gather_descs(1, 3):
        d.wait()
    compute(1)
    start_scatter(1, 3)
    for d in scatter_descs(0, 2):
        d.wait()
    for d in scatter_descs(1, 3):
        d.wait()

    plsc.subcore_barrier()

    db = (hb0.at[pl.ds(0, DCH)], hb1.at[pl.ds(0, DCH)])
    dd = (wb0.at[pl.ds(0, DCH)], wb1.at[pl.ds(0, DCH)])
    for t in range(DRAIN // DCH):
        b = t % 2
        base = pl.multiple_of(s * DRAIN + t * DCH, DCH)
        pltpu.sync_copy(acc.at[pl.ds(base, DCH)], db[b])
        pltpu.sync_copy(db[b], num_ref.at[c].at[pl.ds(base, DCH)])
        pltpu.sync_copy(dacc.at[pl.ds(base, DCH)], dd[b])
        pltpu.sync_copy(dd[b], den_ref.at[c].at[pl.ds(base, DCH)])


def _edge_sc(split, H, ch, h, asn, adn, em):
    D = H * ch
    Dp = D if split == "edge" else D // 2
    if split == "edge":
        head_of_block = [min(j * 16 // ch, H - 1) for j in range(Dp // 16)]
        n_chunks = EROWSP // 2 // NSUB
    else:
        head_of_block = [j * 16 // ch for j in range(Dp // 16)]
        n_chunks = EROWSP // NSUB
    body = functools.partial(_edge_sc_body, split, Dp, head_of_block, n_chunks)
    mesh = plsc.VectorSubcoreMesh(core_axis_name="c", subcore_axis_name="s")
    kfn = pl.kernel(
        body,
        out_type=[
            jax.ShapeDtypeStruct((2, NP, Dp), jnp.float32),
            jax.ShapeDtypeStruct((2, NP, 16), jnp.float32),
        ],
        mesh=mesh,
        compiler_params=pltpu.CompilerParams(use_tc_tiling_on_sc=False),
        scratch_types=(
            [pltpu.VMEM((2, CH), jnp.int32)] * 4
            + [pltpu.VMEM((CH, 16), jnp.float32)] * 4
            + [pltpu.VMEM((CH, Dp), jnp.float32)] * 2
            + [pltpu.VMEM((CH, 16), jnp.float32)] * 2
            + [
                pltpu.VMEM_SHARED((NP, Dp), jnp.float32),
                pltpu.VMEM_SHARED((NP, 16), jnp.float32),
            ]
            + [pltpu.SemaphoreType.DMA] * 14
        ),
    )
    return kfn(em, h, asn, adn)


def _build_ac(a_s, a_d, D):
    H, ch = a_s.shape
    A = jnp.zeros((D, 16), jnp.float32)
    for hh in range(H):
        A = A.at[hh * ch:(hh + 1) * ch, hh].set(a_s[hh])
        A = A.at[hh * ch:(hh + 1) * ch, 8 + hh].set(a_d[hh])
    return A


def kernel(x, edge_index, W1, a1s, a1d, b1, g1, be1, W2, a2s, a2d, b2, g2, be2,
           W3, a3s, a3d, b3):
    src = edge_index[0]
    dst = edge_index[1]
    pad = N + jnp.arange(E_PAD - E, dtype=jnp.int32) % (NP - N)
    srcp = jnp.concatenate([src, pad]).reshape(EROWSP, CH)
    dstp = jnp.concatenate([dst, pad]).reshape(EROWSP, CH)
    em = jnp.stack([srcp, dstp], axis=1)

    h1, aa1 = _dense(x, W1, _build_ac(a1s, a1d, 128))
    asn = jnp.pad(aa1[:, 0:8], ((0, 0), (0, 8)))
    adn = jnp.pad(aa1[:, 8:16], ((0, 0), (0, 8)))
    num, den = _edge_sc("edge", 8, 16, h1, asn, adn, em)
    out1, st1 = _combine("edge", 8, 16, num, den, h1, aa1, b1)

    h2, aa2 = _dense(out1, W2, _build_ac(a2s, a2d, 256), st1, g1, be1)
    h2s = jnp.stack([h2[:, 0:128], h2[:, 128:256]])
    asn2 = jnp.stack([jnp.pad(aa2[:, 0:4], ((0, 0), (0, 12))),
                      jnp.pad(aa2[:, 4:8], ((0, 0), (0, 12)))])
    adn2 = jnp.stack([jnp.pad(aa2[:, 8:12], ((0, 0), (0, 12))),
                      jnp.pad(aa2[:, 12:16], ((0, 0), (0, 12)))])
    num, den = _edge_sc("chan", 8, 32, h2s, asn2, adn2, em)
    out2, st2 = _combine("chan", 8, 32, num, den, h2, aa2, b2)

    h3, aa3 = _dense(out2, W3, _build_ac(a3s, a3d, 16), st2, g2, be2)
    asn = jnp.pad(aa3[:, 0:1], ((0, 0), (0, 15)))
    adn = jnp.pad(aa3[:, 8:9], ((0, 0), (0, 15)))
    num, den = _edge_sc("edge", 1, 16, h3, asn, adn, em)
    return _final(num, den, h3, aa3, b3)
